# baseline (speedup 1.0000x reference)
import functools

import jax
import jax.numpy as jnp
from jax import lax
from jax.experimental import pallas as pl
from jax.experimental.pallas import tpu as pltpu
from jax.experimental.pallas import tpu_sc as plsc

K = 64
N = 32768
ROWS = 128
L = 16
VPS = 8
SB = N // (VPS * L)
NC = 2
NS = 16
NW = NC * NS
RPW = ROWS // NW


def _neg():
    return jnp.full((L,), -jnp.inf, jnp.float32)


def _sortd(v):
    return plsc.sort_key_val(v, v, descending=True)[0]


def _merge2(a, b):
    rb = lax.rev(b, (0,))
    hi = jnp.maximum(a, rb)
    lo = jnp.minimum(a, rb)
    return _sortd(hi), _sortd(lo)


def _insert(s, v):
    s0, s1, s2, s3 = s
    rv = lax.rev(_sortd(v), (0,))
    t3 = _sortd(jnp.maximum(s3, rv))
    s2, t3 = _merge2(s2, t3)
    s1, s2 = _merge2(s1, s2)
    s0, s1 = _merge2(s0, s1)
    return (s0, s1, s2, t3)


def _top4_update(a, v):
    a0, a1, a2, a3 = a
    h0 = jnp.maximum(a0, v)
    n = jnp.minimum(a0, v)
    h1 = jnp.maximum(a1, n)
    n = jnp.minimum(a1, n)
    h2 = jnp.maximum(a2, n)
    n = jnp.minimum(a2, n)
    h3 = jnp.maximum(a3, n)
    return (h0, h1, h2, h3)


def _row_topk(rowbuf, sbuf, cbuf):
    z = _neg()

    @plsc.parallel_loop(0, SB, unroll=4)
    def _p1(i):
        base = i * (VPS * L)
        m = rowbuf[pl.ds(base, L)]
        for j in range(1, VPS):
            m = jnp.maximum(m, rowbuf[pl.ds(base + j * L, L)])
        sbuf[pl.ds(i * L, L)] = m

    def p1b(i, accs):
        return tuple(
            _top4_update(accs[q], sbuf[pl.ds((i * 4 + q) * L, L)])
            for q in range(4)
        )

    accs = lax.fori_loop(0, SB // 4, p1b,
                         tuple((z, z, z, z) for _ in range(4)))
    a = accs[0]
    for q in range(1, 4):
        for v in accs[q]:
            a = _top4_update(a, v)
    t = _sortd(a[3])[15]

    def p2(i, c):
        smax = sbuf[pl.ds(i * L, L)]

        def scan_sub(cc):
            for j in range(VPS):
                v = rowbuf[pl.ds(i * (VPS * L) + j * L, L)]
                mask = v >= t
                plsc.store_compressed(cbuf.at[pl.ds(cc, L)], v, mask=mask)
                cc = cc + plsc.all_reduce_population_count(mask)[0]
            return cc

        return lax.cond(jnp.any(smax >= t), scan_sub, lambda cc: cc, c)

    c = lax.fori_loop(0, SB, p2, jnp.int32(0))

    cbuf[pl.ds(c, L)] = z
    nv = (c + (L - 1)) // L

    def fin(i, s):
        return _insert(s, cbuf[pl.ds(i * L, L)])

    return lax.fori_loop(0, nv, fin, (z, z, z, z))


def _make_sc_kernel():
    mesh = plsc.VectorSubcoreMesh(core_axis_name="c", subcore_axis_name="s")

    @functools.partial(
        pl.kernel,
        mesh=mesh,
        out_type=jax.ShapeDtypeStruct((ROWS, K), jnp.float32),
        compiler_params=pltpu.CompilerParams(needs_layout_passes=False),
        scratch_types=[
            pltpu.VMEM((2 * N,), jnp.float32),
            pltpu.VMEM((SB * L,), jnp.float32),
            pltpu.VMEM((N + L,), jnp.float32),
            pltpu.VMEM((RPW, K), jnp.float32),
            pltpu.SemaphoreType.DMA,
            pltpu.SemaphoreType.DMA,
        ],
    )
    def sc_topk(in_hbm, out_hbm, rowbuf2, sbuf, cbuf, obuf, sem0, sem1):
        wid = lax.axis_index("s") * NC + lax.axis_index("c")
        row0 = wid * RPW
        sems = (sem0, sem1)

        copies = [None, None]
        copies[0] = pltpu.async_copy(in_hbm.at[row0], rowbuf2.at[pl.ds(0, N)],
                                     sem0)
        for r in range(RPW):
            b = r % 2
            copies[b].wait()
            if r + 1 < RPW:
                nb = (r + 1) % 2
                copies[nb] = pltpu.async_copy(
                    in_hbm.at[row0 + r + 1], rowbuf2.at[pl.ds(nb * N, N)],
                    sems[nb])
            s = _row_topk(rowbuf2.at[pl.ds(b * N, N)], sbuf, cbuf)
            for blk in range(4):
                obuf[r, pl.ds(blk * L, L)] = s[blk]
        pltpu.sync_copy(obuf, out_hbm.at[pl.ds(row0, RPW)])

    return sc_topk


_sc_topk = _make_sc_kernel()


@jax.jit
def kernel(input):
    return _sc_topk(input)


# speedup vs baseline: 1.9547x; 1.9547x over previous
import functools

import jax
import jax.numpy as jnp
from jax import lax
from jax.experimental import pallas as pl
from jax.experimental.pallas import tpu as pltpu
from jax.experimental.pallas import tpu_sc as plsc

K = 64
N = 32768
ROWS = 128
L = 16
VPG = 32
GSZ = VPG * L
G = N // GSZ
NC = 2
NS = 16
NW = NC * NS
RPW = ROWS // NW


def _neg():
    return jnp.full((L,), -jnp.inf, jnp.float32)


def _sortd(v):
    return plsc.sort_key_val(v, v, descending=True)[0]


def _merge2(a, b):
    rb = lax.rev(b, (0,))
    hi = jnp.maximum(a, rb)
    lo = jnp.minimum(a, rb)
    return _sortd(hi), _sortd(lo)


def _insert(s, v):
    s0, s1, s2, s3 = s
    rv = lax.rev(_sortd(v), (0,))
    t3 = _sortd(jnp.maximum(s3, rv))
    s2, t3 = _merge2(s2, t3)
    s1, s2 = _merge2(s1, s2)
    s0, s1 = _merge2(s0, s1)
    return (s0, s1, s2, t3)


def _top4_update(a, v):
    a0, a1, a2, a3 = a
    h0 = jnp.maximum(a0, v)
    n = jnp.minimum(a0, v)
    h1 = jnp.maximum(a1, n)
    n = jnp.minimum(a1, n)
    h2 = jnp.maximum(a2, n)
    n = jnp.minimum(a2, n)
    h3 = jnp.maximum(a3, n)
    return (h0, h1, h2, h3)


def _row_topk(rowbuf, gbuf, idbuf, cbuf):
    z = _neg()

    @plsc.parallel_loop(0, G, unroll=2)
    def _p1(g):
        base = g * GSZ
        m = rowbuf[pl.ds(base, L)]
        for j in range(1, VPG):
            m = jnp.maximum(m, rowbuf[pl.ds(base + j * L, L)])
        gbuf[pl.ds(g * L, L)] = m

    def p1b(i, accs):
        return tuple(
            _top4_update(accs[q], gbuf[pl.ds((i * 2 + q) * L, L)])
            for q in range(2)
        )

    accs = lax.fori_loop(0, G // 2, p1b,
                         tuple((z, z, z, z) for _ in range(2)))
    a = accs[0]
    for v in accs[1]:
        a = _top4_update(a, v)
    t = _sortd(a[3])[15]

    iota = jax.lax.broadcasted_iota(jnp.int32, (L,), 0)

    def p2a(i, c):
        mask = gbuf[pl.ds(i * L, L)] >= t
        plsc.store_compressed(idbuf.at[pl.ds(c, L)], iota + i * L, mask=mask)
        return c + plsc.all_reduce_population_count(mask)[0]

    ncell = lax.fori_loop(0, G, p2a, jnp.int32(0))

    def p2b(ci, c):
        cid = idbuf[pl.ds(ci, L)][0]
        base = jax.lax.shift_right_logical(cid, 4) * GSZ \
            + jnp.bitwise_and(cid, L - 1)
        for half in range(2):
            idx = base + half * (L * L) + iota * L
            gth = plsc.load_gather(rowbuf, [idx])
            mask = gth >= t
            plsc.store_compressed(cbuf.at[pl.ds(c, L)], gth, mask=mask)
            c = c + plsc.all_reduce_population_count(mask)[0]
        return c

    c = lax.fori_loop(0, ncell, p2b, jnp.int32(0))

    cbuf[pl.ds(c, L)] = z
    nv = (c + (L - 1)) // L

    def fin(i, s):
        return _insert(s, cbuf[pl.ds(i * L, L)])

    return lax.fori_loop(0, nv, fin, (z, z, z, z))


def _make_sc_kernel():
    mesh = plsc.VectorSubcoreMesh(core_axis_name="c", subcore_axis_name="s")

    @functools.partial(
        pl.kernel,
        mesh=mesh,
        out_type=jax.ShapeDtypeStruct((ROWS, K), jnp.float32),
        compiler_params=pltpu.CompilerParams(needs_layout_passes=False),
        scratch_types=[
            pltpu.VMEM((2 * N,), jnp.float32),
            pltpu.VMEM((G * L,), jnp.float32),
            pltpu.VMEM((G * L + L,), jnp.int32),
            pltpu.VMEM((N + L,), jnp.float32),
            pltpu.VMEM((RPW, K), jnp.float32),
            pltpu.SemaphoreType.DMA,
            pltpu.SemaphoreType.DMA,
        ],
    )
    def sc_topk(in_hbm, out_hbm, rowbuf2, gbuf, idbuf, cbuf, obuf, sem0, sem1):
        wid = lax.axis_index("s") * NC + lax.axis_index("c")
        row0 = wid * RPW
        sems = (sem0, sem1)

        copies = [None, None]
        copies[0] = pltpu.async_copy(in_hbm.at[row0], rowbuf2.at[pl.ds(0, N)],
                                     sem0)
        for r in range(RPW):
            b = r % 2
            copies[b].wait()
            if r + 1 < RPW:
                nb = (r + 1) % 2
                copies[nb] = pltpu.async_copy(
                    in_hbm.at[row0 + r + 1], rowbuf2.at[pl.ds(nb * N, N)],
                    sems[nb])
            s = _row_topk(rowbuf2.at[pl.ds(b * N, N)], gbuf, idbuf, cbuf)
            for blk in range(4):
                obuf[r, pl.ds(blk * L, L)] = s[blk]
        pltpu.sync_copy(obuf, out_hbm.at[pl.ds(row0, RPW)])

    return sc_topk


_sc_topk = _make_sc_kernel()


@jax.jit
def kernel(input):
    return _sc_topk(input)


# speedup vs baseline: 2.4687x; 1.2630x over previous
import functools

import jax
import jax.numpy as jnp
from jax import lax
from jax.experimental import pallas as pl
from jax.experimental.pallas import tpu as pltpu
from jax.experimental.pallas import tpu_sc as plsc

K = 64
N = 32768
ROWS = 128
L = 16
VPG = 32
GSZ = VPG * L
G = N // GSZ
NC = 2
NS = 16
NW = NC * NS
RPW = ROWS // NW


def _neg():
    return jnp.full((L,), -jnp.inf, jnp.float32)


def _sortd(v):
    return plsc.sort_key_val(v, v, descending=True)[0]


def _merge2(a, b):
    rb = lax.rev(b, (0,))
    hi = jnp.maximum(a, rb)
    lo = jnp.minimum(a, rb)
    return _sortd(hi), _sortd(lo)


def _insert(s, v):
    s0, s1, s2, s3 = s
    rv = lax.rev(_sortd(v), (0,))
    t3 = _sortd(jnp.maximum(s3, rv))
    s2, t3 = _merge2(s2, t3)
    s1, s2 = _merge2(s1, s2)
    s0, s1 = _merge2(s0, s1)
    return (s0, s1, s2, t3)


def _top4_update(a, v):
    a0, a1, a2, a3 = a
    h0 = jnp.maximum(a0, v)
    n = jnp.minimum(a0, v)
    h1 = jnp.maximum(a1, n)
    n = jnp.minimum(a1, n)
    h2 = jnp.maximum(a2, n)
    n = jnp.minimum(a2, n)
    h3 = jnp.maximum(a3, n)
    return (h0, h1, h2, h3)


def _row_topk(rowbuf, gbuf, idbuf, vbuf, id2buf, cbuf):
    z = _neg()

    @plsc.parallel_loop(0, G, unroll=2)
    def _p1(g):
        base = g * GSZ
        m = rowbuf[pl.ds(base, L)]
        for j in range(1, VPG):
            m = jnp.maximum(m, rowbuf[pl.ds(base + j * L, L)])
        gbuf[pl.ds(g * L, L)] = m

    def p1b(i, accs):
        return tuple(
            _top4_update(accs[q], gbuf[pl.ds((i * 2 + q) * L, L)])
            for q in range(2)
        )

    accs = lax.fori_loop(0, G // 2, p1b,
                         tuple((z, z, z, z) for _ in range(2)))
    a = accs[0]
    for v in accs[1]:
        a = _top4_update(a, v)
    t = _sortd(a[3])[15]

    iota = jax.lax.broadcasted_iota(jnp.int32, (L,), 0)

    def p2a(i, c):
        gmax = gbuf[pl.ds(i * L, L)]
        mask = gmax >= t
        plsc.store_compressed(idbuf.at[pl.ds(c, L)], iota + i * L, mask=mask)
        plsc.store_compressed(vbuf.at[pl.ds(c, L)], gmax, mask=mask)
        return c + plsc.all_reduce_population_count(mask)[0]

    ncell = lax.fori_loop(0, G, p2a, jnp.int32(0))
    vbuf[pl.ds(ncell, L)] = z

    def tref(i, s):
        return _insert(s, vbuf[pl.ds(i * L, L)])

    sref = lax.fori_loop(0, (ncell + (L - 1)) // L, tref, (z, z, z, z))
    t1 = sref[3][15]

    def p2a2(i, c):
        mask = vbuf[pl.ds(i * L, L)] >= t1
        plsc.store_compressed(id2buf.at[pl.ds(c, L)],
                              idbuf[pl.ds(i * L, L)], mask=mask)
        return c + plsc.all_reduce_population_count(mask)[0]

    ncell2 = lax.fori_loop(0, (ncell + (L - 1)) // L, p2a2, jnp.int32(0))

    def p2b(ci, c):
        cid = id2buf[pl.ds(ci, L)][0]
        base = jax.lax.shift_right_logical(cid, 4) * GSZ \
            + jnp.bitwise_and(cid, L - 1)
        for half in range(2):
            idx = base + half * (L * L) + iota * L
            gth = plsc.load_gather(rowbuf, [idx])
            mask = gth >= t1
            plsc.store_compressed(cbuf.at[pl.ds(c, L)], gth, mask=mask)
            c = c + plsc.all_reduce_population_count(mask)[0]
        return c

    c = lax.fori_loop(0, ncell2, p2b, jnp.int32(0))

    cbuf[pl.ds(c, L)] = z
    nv = (c + (L - 1)) // L

    def fin(i, s):
        return _insert(s, cbuf[pl.ds(i * L, L)])

    return lax.fori_loop(0, nv, fin, (z, z, z, z))


def _make_sc_kernel():
    mesh = plsc.VectorSubcoreMesh(core_axis_name="c", subcore_axis_name="s")

    @functools.partial(
        pl.kernel,
        mesh=mesh,
        out_type=jax.ShapeDtypeStruct((ROWS, K), jnp.float32),
        compiler_params=pltpu.CompilerParams(needs_layout_passes=False),
        scratch_types=[
            pltpu.VMEM((2 * N,), jnp.float32),
            pltpu.VMEM((G * L,), jnp.float32),
            pltpu.VMEM((G * L + L,), jnp.int32),
            pltpu.VMEM((G * L + L,), jnp.float32),
            pltpu.VMEM((G * L + L,), jnp.int32),
            pltpu.VMEM((N + L,), jnp.float32),
            pltpu.VMEM((RPW, K), jnp.float32),
            pltpu.SemaphoreType.DMA,
            pltpu.SemaphoreType.DMA,
        ],
    )
    def sc_topk(in_hbm, out_hbm, rowbuf2, gbuf, idbuf, vbuf, id2buf, cbuf,
                obuf, sem0, sem1):
        wid = lax.axis_index("s") * NC + lax.axis_index("c")
        row0 = wid * RPW
        sems = (sem0, sem1)

        copies = [None, None]
        copies[0] = pltpu.async_copy(in_hbm.at[row0], rowbuf2.at[pl.ds(0, N)],
                                     sem0)
        for r in range(RPW):
            b = r % 2
            copies[b].wait()
            if r + 1 < RPW:
                nb = (r + 1) % 2
                copies[nb] = pltpu.async_copy(
                    in_hbm.at[row0 + r + 1], rowbuf2.at[pl.ds(nb * N, N)],
                    sems[nb])
            s = _row_topk(rowbuf2.at[pl.ds(b * N, N)], gbuf, idbuf, vbuf,
                          id2buf, cbuf)
            for blk in range(4):
                obuf[r, pl.ds(blk * L, L)] = s[blk]
        pltpu.sync_copy(obuf, out_hbm.at[pl.ds(row0, RPW)])

    return sc_topk


_sc_topk = _make_sc_kernel()


@jax.jit
def kernel(input):
    return _sc_topk(input)


# speedup vs baseline: 2.6782x; 1.0849x over previous
import functools

import jax
import jax.numpy as jnp
from jax import lax
from jax.experimental import pallas as pl
from jax.experimental.pallas import tpu as pltpu
from jax.experimental.pallas import tpu_sc as plsc

K = 64
N = 32768
ROWS = 128
L = 16
VPG = 32
GSZ = VPG * L
G = N // GSZ
NC = 2
NS = 16
NW = NC * NS
RPW = ROWS // NW


def _neg():
    return jnp.full((L,), -jnp.inf, jnp.float32)


def _sortd(v):
    return plsc.sort_key_val(v, v, descending=True)[0]


def _merge2(a, b):
    rb = lax.rev(b, (0,))
    hi = jnp.maximum(a, rb)
    lo = jnp.minimum(a, rb)
    return _sortd(hi), _sortd(lo)


def _insert(s, v):
    s0, s1, s2, s3 = s
    rv = lax.rev(_sortd(v), (0,))
    t3 = _sortd(jnp.maximum(s3, rv))
    s2, t3 = _merge2(s2, t3)
    s1, s2 = _merge2(s1, s2)
    s0, s1 = _merge2(s0, s1)
    return (s0, s1, s2, t3)


def _top4_update(a, v):
    a0, a1, a2, a3 = a
    h0 = jnp.maximum(a0, v)
    n = jnp.minimum(a0, v)
    h1 = jnp.maximum(a1, n)
    n = jnp.minimum(a1, n)
    h2 = jnp.maximum(a2, n)
    n = jnp.minimum(a2, n)
    h3 = jnp.maximum(a3, n)
    return (h0, h1, h2, h3)


def _row_topk(rowbuf, gbuf, idbuf, vbuf, id2buf, cbuf):
    z = _neg()

    @plsc.parallel_loop(0, G, unroll=2)
    def _p1(g):
        base = g * GSZ
        m = rowbuf[pl.ds(base, L)]
        for j in range(1, VPG):
            m = jnp.maximum(m, rowbuf[pl.ds(base + j * L, L)])
        gbuf[pl.ds(g * L, L)] = m

    def p1b(i, accs):
        return tuple(
            _top4_update(accs[q], gbuf[pl.ds((i * 2 + q) * L, L)])
            for q in range(2)
        )

    accs = lax.fori_loop(0, G // 2, p1b,
                         tuple((z, z, z, z) for _ in range(2)))
    a = accs[0]
    for v in accs[1]:
        a = _top4_update(a, v)
    t = _sortd(a[3])[15]

    iota = jax.lax.broadcasted_iota(jnp.int32, (L,), 0)

    def p2a(i, c):
        gmax = gbuf[pl.ds(i * L, L)]
        mask = gmax >= t
        plsc.store_compressed(idbuf.at[pl.ds(c, L)], iota + i * L, mask=mask)
        plsc.store_compressed(vbuf.at[pl.ds(c, L)], gmax, mask=mask)
        return c + plsc.all_reduce_population_count(mask)[0]

    ncell = lax.fori_loop(0, G, p2a, jnp.int32(0))
    vbuf[pl.ds(ncell, L)] = z

    def tref(i, s):
        return _insert(s, vbuf[pl.ds(i * L, L)])

    sref = lax.fori_loop(0, (ncell + (L - 1)) // L, tref, (z, z, z, z))
    t1 = sref[3][15]

    def p2a2(i, c):
        mask = vbuf[pl.ds(i * L, L)] >= t1
        plsc.store_compressed(id2buf.at[pl.ds(c, L)],
                              idbuf[pl.ds(i * L, L)], mask=mask)
        return c + plsc.all_reduce_population_count(mask)[0]

    ncell2 = lax.fori_loop(0, (ncell + (L - 1)) // L, p2a2, jnp.int32(0))

    def p2b(bi, c):
        ids = jnp.bitwise_and(id2buf[pl.ds(bi * L, L)], G * L - 1)
        valid = (bi * L + iota) < ncell2
        bases = jax.lax.shift_right_logical(ids, 4) * GSZ \
            + jnp.bitwise_and(ids, L - 1)
        for j in range(VPG):
            gth = plsc.load_gather(rowbuf, [bases + j * L], mask=valid)
            mask = jnp.logical_and(gth >= t1, valid)
            plsc.store_compressed(cbuf.at[pl.ds(c, L)], gth, mask=mask)
            c = c + plsc.all_reduce_population_count(mask)[0]
        return c

    c = lax.fori_loop(0, (ncell2 + (L - 1)) // L, p2b, jnp.int32(0))

    cbuf[pl.ds(c, L)] = z
    nv = (c + (L - 1)) // L

    def fin(i, s):
        return _insert(s, cbuf[pl.ds(i * L, L)])

    return lax.fori_loop(0, nv, fin, (z, z, z, z))


def _make_sc_kernel():
    mesh = plsc.VectorSubcoreMesh(core_axis_name="c", subcore_axis_name="s")

    @functools.partial(
        pl.kernel,
        mesh=mesh,
        out_type=jax.ShapeDtypeStruct((ROWS, K), jnp.float32),
        compiler_params=pltpu.CompilerParams(needs_layout_passes=False),
        scratch_types=[
            pltpu.VMEM((2 * N,), jnp.float32),
            pltpu.VMEM((G * L,), jnp.float32),
            pltpu.VMEM((G * L + L,), jnp.int32),
            pltpu.VMEM((G * L + L,), jnp.float32),
            pltpu.VMEM((G * L + L,), jnp.int32),
            pltpu.VMEM((N + L,), jnp.float32),
            pltpu.VMEM((RPW, K), jnp.float32),
            pltpu.SemaphoreType.DMA,
            pltpu.SemaphoreType.DMA,
        ],
    )
    def sc_topk(in_hbm, out_hbm, rowbuf2, gbuf, idbuf, vbuf, id2buf, cbuf,
                obuf, sem0, sem1):
        wid = lax.axis_index("s") * NC + lax.axis_index("c")
        row0 = wid * RPW
        sems = (sem0, sem1)

        copies = [None, None]
        copies[0] = pltpu.async_copy(in_hbm.at[row0], rowbuf2.at[pl.ds(0, N)],
                                     sem0)
        for r in range(RPW):
            b = r % 2
            copies[b].wait()
            if r + 1 < RPW:
                nb = (r + 1) % 2
                copies[nb] = pltpu.async_copy(
                    in_hbm.at[row0 + r + 1], rowbuf2.at[pl.ds(nb * N, N)],
                    sems[nb])
            s = _row_topk(rowbuf2.at[pl.ds(b * N, N)], gbuf, idbuf, vbuf,
                          id2buf, cbuf)
            for blk in range(4):
                obuf[r, pl.ds(blk * L, L)] = s[blk]
        pltpu.sync_copy(obuf, out_hbm.at[pl.ds(row0, RPW)])

    return sc_topk


_sc_topk = _make_sc_kernel()


@jax.jit
def kernel(input):
    return _sc_topk(input)


# speedup vs baseline: 2.6822x; 1.0015x over previous
import functools

import jax
import jax.numpy as jnp
from jax import lax
from jax.experimental import pallas as pl
from jax.experimental.pallas import tpu as pltpu
from jax.experimental.pallas import tpu_sc as plsc

K = 64
N = 32768
ROWS = 128
L = 16
VPG = 32
GSZ = VPG * L
G = N // GSZ
NC = 2
NS = 16
NW = NC * NS
RPW = ROWS // NW


def _neg():
    return jnp.full((L,), -jnp.inf, jnp.float32)


def _sortd(v):
    return plsc.sort_key_val(v, v, descending=True)[0]


def _merge2(a, b):
    rb = lax.rev(b, (0,))
    hi = jnp.maximum(a, rb)
    lo = jnp.minimum(a, rb)
    return _sortd(hi), _sortd(lo)


def _insert(s, v):
    s0, s1, s2, s3 = s
    rv = lax.rev(_sortd(v), (0,))
    t3 = _sortd(jnp.maximum(s3, rv))
    s2, t3 = _merge2(s2, t3)
    s1, s2 = _merge2(s1, s2)
    s0, s1 = _merge2(s0, s1)
    return (s0, s1, s2, t3)


def _top4_update(a, v):
    a0, a1, a2, a3 = a
    h0 = jnp.maximum(a0, v)
    n = jnp.minimum(a0, v)
    h1 = jnp.maximum(a1, n)
    n = jnp.minimum(a1, n)
    h2 = jnp.maximum(a2, n)
    n = jnp.minimum(a2, n)
    h3 = jnp.maximum(a3, n)
    return (h0, h1, h2, h3)


def _row_topk(rowbuf, gbuf, idbuf, vbuf, id2buf, cbuf):
    z = _neg()

    def p1(g, acc):
        base = g * GSZ
        m = rowbuf[pl.ds(base, L)]
        for j in range(1, VPG):
            m = jnp.maximum(m, rowbuf[pl.ds(base + j * L, L)])
        gbuf[pl.ds(g * L, L)] = m
        return _top4_update(acc, m)

    a = plsc.parallel_loop(0, G, unroll=2, carry=(z, z, z, z))(p1)
    t = _sortd(a[3])[15]

    iota = jax.lax.broadcasted_iota(jnp.int32, (L,), 0)

    def p2a(i, c):
        gmax = gbuf[pl.ds(i * L, L)]
        mask = gmax >= t
        plsc.store_compressed(idbuf.at[pl.ds(c, L)], iota + i * L, mask=mask)
        plsc.store_compressed(vbuf.at[pl.ds(c, L)], gmax, mask=mask)
        return c + plsc.all_reduce_population_count(mask)[0]

    ncell = lax.fori_loop(0, G, p2a, jnp.int32(0))
    vbuf[pl.ds(ncell, L)] = z

    def tref(i, s):
        return _insert(s, vbuf[pl.ds(i * L, L)])

    sref = lax.fori_loop(0, (ncell + (L - 1)) // L, tref, (z, z, z, z))
    t1 = sref[3][15]

    def p2a2(i, c):
        mask = vbuf[pl.ds(i * L, L)] >= t1
        plsc.store_compressed(id2buf.at[pl.ds(c, L)],
                              idbuf[pl.ds(i * L, L)], mask=mask)
        return c + plsc.all_reduce_population_count(mask)[0]

    ncell2 = lax.fori_loop(0, (ncell + (L - 1)) // L, p2a2, jnp.int32(0))

    def p2b(bi, c):
        ids = jnp.bitwise_and(id2buf[pl.ds(bi * L, L)], G * L - 1)
        valid = (bi * L + iota) < ncell2
        bases = jax.lax.shift_right_logical(ids, 4) * GSZ \
            + jnp.bitwise_and(ids, L - 1)
        for j in range(VPG):
            gth = plsc.load_gather(rowbuf, [bases + j * L], mask=valid)
            mask = jnp.logical_and(gth >= t1, valid)
            plsc.store_compressed(cbuf.at[pl.ds(c, L)], gth, mask=mask)
            c = c + plsc.all_reduce_population_count(mask)[0]
        return c

    c = lax.fori_loop(0, (ncell2 + (L - 1)) // L, p2b, jnp.int32(0))

    cbuf[pl.ds(c, L)] = z
    nv = (c + (L - 1)) // L

    def fin(i, s):
        return _insert(s, cbuf[pl.ds(i * L, L)])

    return lax.fori_loop(0, nv, fin, (z, z, z, z))


def _make_sc_kernel():
    mesh = plsc.VectorSubcoreMesh(core_axis_name="c", subcore_axis_name="s")

    @functools.partial(
        pl.kernel,
        mesh=mesh,
        out_type=jax.ShapeDtypeStruct((ROWS, K), jnp.float32),
        compiler_params=pltpu.CompilerParams(needs_layout_passes=False),
        scratch_types=[
            pltpu.VMEM((2 * N,), jnp.float32),
            pltpu.VMEM((G * L,), jnp.float32),
            pltpu.VMEM((G * L + L,), jnp.int32),
            pltpu.VMEM((G * L + L,), jnp.float32),
            pltpu.VMEM((G * L + L,), jnp.int32),
            pltpu.VMEM((N + L,), jnp.float32),
            pltpu.VMEM((RPW, K), jnp.float32),
            pltpu.SemaphoreType.DMA,
            pltpu.SemaphoreType.DMA,
        ],
    )
    def sc_topk(in_hbm, out_hbm, rowbuf2, gbuf, idbuf, vbuf, id2buf, cbuf,
                obuf, sem0, sem1):
        wid = lax.axis_index("s") * NC + lax.axis_index("c")
        row0 = wid * RPW
        sems = (sem0, sem1)

        copies = [None, None]
        copies[0] = pltpu.async_copy(in_hbm.at[row0], rowbuf2.at[pl.ds(0, N)],
                                     sem0)
        for r in range(RPW):
            b = r % 2
            copies[b].wait()
            if r + 1 < RPW:
                nb = (r + 1) % 2
                copies[nb] = pltpu.async_copy(
                    in_hbm.at[row0 + r + 1], rowbuf2.at[pl.ds(nb * N, N)],
                    sems[nb])
            s = _row_topk(rowbuf2.at[pl.ds(b * N, N)], gbuf, idbuf, vbuf,
                          id2buf, cbuf)
            for blk in range(4):
                obuf[r, pl.ds(blk * L, L)] = s[blk]
        pltpu.sync_copy(obuf, out_hbm.at[pl.ds(row0, RPW)])

    return sc_topk


_sc_topk = _make_sc_kernel()


@jax.jit
def kernel(input):
    return _sc_topk(input)
